# baseline (speedup 1.0000x reference)
import jax
import jax.numpy as jnp
from jax.experimental import pallas as pl
from jax.experimental.pallas import tpu as pltpu

_HALF = 1024
_BR = 512


def _mwn_kernel(x_ref, w_ref, o_ref, scr, sem):
    i = pl.program_id(0)
    nsteps = pl.num_programs(0)

    xb = x_ref[...]
    q = xb[:, :_HALF]
    y = xb[:, _HALF:]
    p = q * y
    z1 = 1.0 - jnp.sum(p, axis=1, keepdims=True)
    z2 = p * w_ref[...]
    m = jnp.maximum(jnp.max(z2, axis=1, keepdims=True), z1)
    e1 = jnp.exp(z1 - m)
    e2 = jnp.exp(z2 - m)
    r = 1.0 / (e1 + jnp.sum(e2, axis=1, keepdims=True))
    scr[pl.ds(i * _BR, _BR), 0:1] = e1 * r
    scr[pl.ds(i * _BR, _BR), 1:_HALF + 1] = e2 * r

    @pl.when(i == nsteps - 1)
    def _():
        cp = pltpu.make_async_copy(scr, o_ref, sem)
        cp.start()
        cp.wait()


def kernel(x, weights):
    n = x.shape[0]
    w2d = weights.reshape(1, _HALF)
    grid = (n // _BR,)
    return pl.pallas_call(
        _mwn_kernel,
        grid=grid,
        in_specs=[
            pl.BlockSpec((_BR, 2 * _HALF), lambda i: (i, 0)),
            pl.BlockSpec((1, _HALF), lambda i: (0, 0)),
        ],
        out_specs=pl.BlockSpec(memory_space=pltpu.MemorySpace.HBM),
        out_shape=jax.ShapeDtypeStruct((n, _HALF + 1), jnp.float32),
        scratch_shapes=[
            pltpu.VMEM((n, _HALF + 1), jnp.float32),
            pltpu.SemaphoreType.DMA,
        ],
        compiler_params=pltpu.CompilerParams(
            dimension_semantics=("arbitrary",),
        ),
    )(x, w2d)


# speedup vs baseline: 1.1110x; 1.1110x over previous
import jax
import jax.numpy as jnp
from jax.experimental import pallas as pl
from jax.experimental.pallas import tpu as pltpu

_HALF = 1024
_BR = 512
_NIN = 4
_LOOKAHEAD = 3


def _mwn_kernel(x_ref, w_ref, o_ref, xbuf, obuf, insem, outsem):
    i = pl.program_id(0)
    nsteps = pl.num_programs(0)
    islot = jax.lax.rem(i, _NIN)
    oslot = jax.lax.rem(i, 2)

    def in_copy(step, s):
        return pltpu.make_async_copy(
            x_ref.at[pl.ds(step * _BR, _BR), :],
            xbuf.at[s],
            insem.at[s])

    def out_copies(step, s):
        main = pltpu.make_async_copy(
            obuf.at[s, :, 0:_HALF],
            o_ref.at[pl.ds(step * _BR, _BR), pl.ds(0, _HALF)],
            outsem.at[s, 0])
        tail = pltpu.make_async_copy(
            obuf.at[s, :, _HALF:_HALF + 1],
            o_ref.at[pl.ds(step * _BR, _BR), pl.ds(_HALF, 1)],
            outsem.at[s, 1])
        return main, tail

    @pl.when(i == 0)
    def _():
        for k in range(_LOOKAHEAD):
            in_copy(k, k).start()

    @pl.when(i + _LOOKAHEAD < nsteps)
    def _():
        in_copy(i + _LOOKAHEAD, jax.lax.rem(i + _LOOKAHEAD, _NIN)).start()

    in_copy(i, islot).wait()

    @pl.when(i >= 2)
    def _():
        m, t = out_copies(i - 2, oslot)
        m.wait()
        t.wait()

    xb = xbuf[islot]
    q = xb[:, :_HALF]
    y = xb[:, _HALF:]
    p = q * y
    z1 = 1.0 - jnp.sum(p, axis=1, keepdims=True)
    z2 = p * w_ref[...]
    m_ = jnp.maximum(jnp.max(z2, axis=1, keepdims=True), z1)
    e1 = jnp.exp(z1 - m_)
    e2 = jnp.exp(z2 - m_)
    r = 1.0 / (e1 + jnp.sum(e2, axis=1, keepdims=True))
    obuf[oslot, :, 0:1] = e1 * r
    obuf[oslot, :, 1:_HALF + 1] = e2 * r

    mc, tc = out_copies(i, oslot)
    mc.start()
    tc.start()

    @pl.when(i == nsteps - 1)
    def _():
        for k in (nsteps - 2, nsteps - 1):
            m, t = out_copies(k, k % 2)
            m.wait()
            t.wait()


def kernel(x, weights):
    n = x.shape[0]
    w2d = weights.reshape(1, _HALF)
    grid = (n // _BR,)
    return pl.pallas_call(
        _mwn_kernel,
        grid=grid,
        in_specs=[
            pl.BlockSpec(memory_space=pltpu.MemorySpace.HBM),
            pl.BlockSpec((1, _HALF), lambda i: (0, 0)),
        ],
        out_specs=pl.BlockSpec(memory_space=pltpu.MemorySpace.HBM),
        out_shape=jax.ShapeDtypeStruct((n, _HALF + 1), jnp.float32),
        scratch_shapes=[
            pltpu.VMEM((_NIN, _BR, 2 * _HALF), jnp.float32),
            pltpu.VMEM((2, _BR, _HALF + 1), jnp.float32),
            pltpu.SemaphoreType.DMA((_NIN,)),
            pltpu.SemaphoreType.DMA((2, 2)),
        ],
        compiler_params=pltpu.CompilerParams(
            dimension_semantics=("arbitrary",),
        ),
    )(x, w2d)


# speedup vs baseline: 1.1143x; 1.0029x over previous
import jax
import jax.numpy as jnp
from jax.experimental import pallas as pl
from jax.experimental.pallas import tpu as pltpu

_HALF = 1024
_BR = 512
_NIN = 6
_LOOKAHEAD = 5


def _mwn_kernel(x_ref, w_ref, o_ref, xbuf, obuf, insem, outsem):
    i = pl.program_id(0)
    nsteps = pl.num_programs(0)
    islot = jax.lax.rem(i, _NIN)
    oslot = jax.lax.rem(i, 3)

    def in_copy(step, s):
        return pltpu.make_async_copy(
            x_ref.at[pl.ds(step * _BR, _BR), :],
            xbuf.at[s],
            insem.at[s])

    def out_copies(step, s):
        main = pltpu.make_async_copy(
            obuf.at[s, :, 0:_HALF],
            o_ref.at[pl.ds(step * _BR, _BR), pl.ds(0, _HALF)],
            outsem.at[s, 0])
        tail = pltpu.make_async_copy(
            obuf.at[s, :, _HALF:_HALF + 1],
            o_ref.at[pl.ds(step * _BR, _BR), pl.ds(_HALF, 1)],
            outsem.at[s, 1])
        return main, tail

    @pl.when(i == 0)
    def _():
        for k in range(_LOOKAHEAD):
            in_copy(k, k).start()

    @pl.when(i + _LOOKAHEAD < nsteps)
    def _():
        in_copy(i + _LOOKAHEAD, jax.lax.rem(i + _LOOKAHEAD, _NIN)).start()

    in_copy(i, islot).wait()

    @pl.when(i >= 3)
    def _():
        m, t = out_copies(i - 3, oslot)
        m.wait()
        t.wait()

    xb = xbuf[islot]
    q = xb[:, :_HALF]
    y = xb[:, _HALF:]
    p = q * y
    z1 = 1.0 - jnp.sum(p, axis=1, keepdims=True)
    z2 = p * w_ref[...]
    m_ = jnp.maximum(jnp.max(z2, axis=1, keepdims=True), z1)
    e1 = jnp.exp(z1 - m_)
    e2 = jnp.exp(z2 - m_)
    r = 1.0 / (e1 + jnp.sum(e2, axis=1, keepdims=True))
    obuf[oslot, :, 0:1] = e1 * r
    obuf[oslot, :, 1:_HALF + 1] = e2 * r

    mc, tc = out_copies(i, oslot)
    mc.start()
    tc.start()

    @pl.when(i == nsteps - 1)
    def _():
        for k in (nsteps - 3, nsteps - 2, nsteps - 1):
            m, t = out_copies(k, k % 3)
            m.wait()
            t.wait()


def kernel(x, weights):
    n = x.shape[0]
    w2d = weights.reshape(1, _HALF)
    grid = (n // _BR,)
    return pl.pallas_call(
        _mwn_kernel,
        grid=grid,
        in_specs=[
            pl.BlockSpec(memory_space=pltpu.MemorySpace.HBM),
            pl.BlockSpec((1, _HALF), lambda i: (0, 0)),
        ],
        out_specs=pl.BlockSpec(memory_space=pltpu.MemorySpace.HBM),
        out_shape=jax.ShapeDtypeStruct((n, _HALF + 1), jnp.float32),
        scratch_shapes=[
            pltpu.VMEM((_NIN, _BR, 2 * _HALF), jnp.float32),
            pltpu.VMEM((3, _BR, _HALF + 1), jnp.float32),
            pltpu.SemaphoreType.DMA((_NIN,)),
            pltpu.SemaphoreType.DMA((3, 2)),
        ],
        compiler_params=pltpu.CompilerParams(
            dimension_semantics=("arbitrary",),
        ),
    )(x, w2d)


# speedup vs baseline: 1.1238x; 1.0085x over previous
import jax
import jax.numpy as jnp
from jax.experimental import pallas as pl
from jax.experimental.pallas import tpu as pltpu

_HALF = 1024
_BR = 256
_NIN = 6
_LOOKAHEAD = 5


def _mwn_kernel(x_ref, w_ref, o_ref, xbuf, obuf, insem, outsem):
    i = pl.program_id(0)
    nsteps = pl.num_programs(0)
    islot = jax.lax.rem(i, _NIN)
    oslot = jax.lax.rem(i, 3)

    def in_copy(step, s):
        return pltpu.make_async_copy(
            x_ref.at[pl.ds(step * _BR, _BR), :],
            xbuf.at[s],
            insem.at[s])

    def out_copies(step, s):
        main = pltpu.make_async_copy(
            obuf.at[s, :, 0:_HALF],
            o_ref.at[pl.ds(step * _BR, _BR), pl.ds(0, _HALF)],
            outsem.at[s, 0])
        tail = pltpu.make_async_copy(
            obuf.at[s, :, _HALF:_HALF + 1],
            o_ref.at[pl.ds(step * _BR, _BR), pl.ds(_HALF, 1)],
            outsem.at[s, 1])
        return main, tail

    @pl.when(i == 0)
    def _():
        for k in range(_LOOKAHEAD):
            in_copy(k, k).start()

    @pl.when(i + _LOOKAHEAD < nsteps)
    def _():
        in_copy(i + _LOOKAHEAD, jax.lax.rem(i + _LOOKAHEAD, _NIN)).start()

    in_copy(i, islot).wait()

    @pl.when(i >= 3)
    def _():
        m, t = out_copies(i - 3, oslot)
        m.wait()
        t.wait()

    xb = xbuf[islot]
    q = xb[:, :_HALF]
    y = xb[:, _HALF:]
    p = q * y
    z1 = 1.0 - jnp.sum(p, axis=1, keepdims=True)
    z2 = p * w_ref[...]
    m_ = jnp.maximum(jnp.max(z2, axis=1, keepdims=True), z1)
    e1 = jnp.exp(z1 - m_)
    e2 = jnp.exp(z2 - m_)
    r = 1.0 / (e1 + jnp.sum(e2, axis=1, keepdims=True))
    obuf[oslot, :, 0:1] = e1 * r
    obuf[oslot, :, 1:_HALF + 1] = e2 * r

    mc, tc = out_copies(i, oslot)
    mc.start()
    tc.start()

    @pl.when(i == nsteps - 1)
    def _():
        for k in (nsteps - 3, nsteps - 2, nsteps - 1):
            m, t = out_copies(k, k % 3)
            m.wait()
            t.wait()


def kernel(x, weights):
    n = x.shape[0]
    w2d = weights.reshape(1, _HALF)
    grid = (n // _BR,)
    return pl.pallas_call(
        _mwn_kernel,
        grid=grid,
        in_specs=[
            pl.BlockSpec(memory_space=pltpu.MemorySpace.HBM),
            pl.BlockSpec((1, _HALF), lambda i: (0, 0)),
        ],
        out_specs=pl.BlockSpec(memory_space=pltpu.MemorySpace.HBM),
        out_shape=jax.ShapeDtypeStruct((n, _HALF + 1), jnp.float32),
        scratch_shapes=[
            pltpu.VMEM((_NIN, _BR, 2 * _HALF), jnp.float32),
            pltpu.VMEM((3, _BR, _HALF + 1), jnp.float32),
            pltpu.SemaphoreType.DMA((_NIN,)),
            pltpu.SemaphoreType.DMA((3, 2)),
        ],
        compiler_params=pltpu.CompilerParams(
            dimension_semantics=("arbitrary",),
        ),
    )(x, w2d)
